# baseline (speedup 1.0000x reference)
import functools

import jax
import jax.numpy as jnp
from jax import lax
from jax.experimental import pallas as pl
from jax.experimental.pallas import tpu as pltpu
from jax.experimental.pallas import tpu_sc as plsc

NC = 2
NS = 16
L = 16
NW = NC * NS
B = 16384
D = 32
BPW = B // NW
NCH = 4
CH = BPW // NCH

_mesh = plsc.VectorSubcoreMesh(core_axis_name="c", subcore_axis_name="s")


@functools.partial(
    pl.kernel,
    out_type=jax.ShapeDtypeStruct((NW, NCH, CH), jnp.float32),
    mesh=_mesh,
    compiler_params=pltpu.CompilerParams(
        needs_layout_passes=False, use_tc_tiling_on_sc=False),
    scratch_types=[
        pltpu.VMEM((NCH, CH), jnp.int32),
        pltpu.VMEM((NCH, CH), jnp.int32),
        pltpu.VMEM((NCH, CH, D), jnp.float32),
        pltpu.VMEM((NCH, CH, D), jnp.float32),
        pltpu.VMEM((NCH, CH), jnp.float32),
        pltpu.VMEM((NCH, CH), jnp.float32),
        pltpu.VMEM((L,), jnp.float32),
        pltpu.VMEM((NCH, CH), jnp.float32),
        pltpu.SemaphoreType.DMA,
    ],
)
def _mf_sc(uid_hbm, iid_hbm, uemb_hbm, iemb_hbm, ubias_hbm, ibias_hbm,
           gb_hbm, out_hbm, uid_v, iid_v, urows_v, irows_v, bu_v, bi_v,
           gb_v, out_v, sem):
    wid = lax.axis_index("s") * NC + lax.axis_index("c")

    pltpu.sync_copy(uid_hbm.at[wid], uid_v)
    pltpu.sync_copy(iid_hbm.at[wid], iid_v)
    pltpu.sync_copy(gb_hbm, gb_v)

    copies = []
    for c in range(NCH):
        copies.append(pltpu.async_copy(uemb_hbm.at[uid_v.at[c]], urows_v.at[c], sem))
        copies.append(pltpu.async_copy(iemb_hbm.at[iid_v.at[c]], irows_v.at[c], sem))
        copies.append(pltpu.async_copy(ubias_hbm.at[uid_v.at[c]], bu_v.at[c], sem))
        copies.append(pltpu.async_copy(ibias_hbm.at[iid_v.at[c]], bi_v.at[c], sem))
    for cp in copies:
        cp.wait()

    gb = gb_v[:]
    lane = lax.iota(jnp.int32, L)

    for c in range(NCH):
        def group_body(g, carry, c=c):
            base = g * L
            acc = jnp.zeros((L,), jnp.float32)
            for j in range(L):
                r = base + j
                u0 = urows_v[c, r, pl.ds(0, L)]
                u1 = urows_v[c, r, pl.ds(L, L)]
                i0 = irows_v[c, r, pl.ds(0, L)]
                i1 = irows_v[c, r, pl.ds(L, L)]
                s = jnp.sum(u0 * i0 + u1 * i1)
                acc = jnp.where(lane == j, s, acc)
            out_v[c, pl.ds(base, L)] = (
                acc + bu_v[c, pl.ds(base, L)] + bi_v[c, pl.ds(base, L)] + gb
            )
            return carry
        lax.fori_loop(0, CH // L, group_body, 0)

    pltpu.sync_copy(out_v, out_hbm.at[wid])


def kernel(user_id, item_id, user_embedding, item_embedding, user_bias,
           item_bias, global_bias):
    uid = user_id.astype(jnp.int32).reshape(NW, NCH, CH)
    iid = item_id.astype(jnp.int32).reshape(NW, NCH, CH)
    gb16 = jnp.broadcast_to(global_bias.astype(jnp.float32), (L,))
    out = _mf_sc(uid, iid, user_embedding, item_embedding, user_bias,
                 item_bias, gb16)
    return out.reshape(B)
